# Initial kernel scaffold; baseline (speedup 1.0000x reference)
import jax
import jax.numpy as jnp
from jax.experimental import pallas as pl


def kernel(x, edge_index, edge_attr, batch_index, mol_fingerprints, W1, b1, W2, b2, LW1, LB1, LW2, LB2):
    raise NotImplementedError("write your pallas kernel here")


# speedup vs baseline: 18.1109x; 18.1109x over previous
import functools

import jax
import jax.numpy as jnp
from jax import lax
from jax.experimental import pallas as pl
from jax.experimental.pallas import tpu as pltpu
from jax.experimental.pallas import tpu_sc as plsc

N = 10000
E = 320000
C = 64
NP = 10240
K = 128
NCHUNK = E // K
NCORES = 2
NSUB = 16
NW = NCORES * NSUB
CPW = (NCHUNK + NW - 1) // NW
RPS = NP // NSUB
DC = 16


def _sc_mesh():
    return plsc.VectorSubcoreMesh(
        core_axis_name="c", subcore_axis_name="s",
        num_cores=NCORES, num_subcores=NSUB)


def _sc_degree(dst, zeros16):
    @functools.partial(
        pl.kernel,
        out_type=jax.ShapeDtypeStruct((NCORES, NP, DC), jnp.float32),
        mesh=_sc_mesh(),
        scratch_types=[
            pltpu.VMEM((K,), jnp.int32),
            pltpu.VMEM((K, DC), jnp.float32),
            pltpu.VMEM_SHARED((NP, DC), jnp.float32),
        ],
        compiler_params=pltpu.CompilerParams(use_tc_tiling_on_sc=False),
    )
    def deg_kernel(dst_hbm, zeros_hbm, out_hbm, didx, ones, acc):
        cid = lax.axis_index("c")
        sid = lax.axis_index("s")
        wid = sid * NCORES + cid
        r0 = sid * RPS

        @pl.loop(0, K)
        def _(i):
            ones[i, :] = jnp.full((DC,), 1.0, jnp.float32)

        pltpu.sync_copy(zeros_hbm.at[pl.ds(r0, RPS)], acc.at[pl.ds(r0, RPS)])
        plsc.subcore_barrier()

        @pl.loop(0, CPW)
        def _(j):
            cix = wid + NW * j

            @pl.when(cix < NCHUNK)
            def _():
                off = pl.multiple_of(cix * K, K)
                pltpu.sync_copy(dst_hbm.at[pl.ds(off, K)], didx)
                pltpu.sync_copy(ones, acc.at[didx], add=True)

        plsc.subcore_barrier()
        pltpu.sync_copy(acc.at[pl.ds(r0, RPS)], out_hbm.at[cid, pl.ds(r0, RPS)])

    return deg_kernel(dst, zeros16)


def _sc_propagate(hp, src, dst, zeros64):
    @functools.partial(
        pl.kernel,
        out_type=jax.ShapeDtypeStruct((NCORES, NP, C), jnp.float32),
        mesh=_sc_mesh(),
        scratch_types=[
            pltpu.VMEM((K,), jnp.int32),
            pltpu.VMEM((K,), jnp.int32),
            pltpu.VMEM((K, C), jnp.float32),
            pltpu.VMEM_SHARED((NP, C), jnp.float32),
        ],
        compiler_params=pltpu.CompilerParams(use_tc_tiling_on_sc=False),
    )
    def prop_kernel(hp_hbm, src_hbm, dst_hbm, zeros_hbm, out_hbm,
                    sidx, didx, rows, acc):
        cid = lax.axis_index("c")
        sid = lax.axis_index("s")
        wid = sid * NCORES + cid
        r0 = sid * RPS

        pltpu.sync_copy(zeros_hbm.at[pl.ds(r0, RPS)], acc.at[pl.ds(r0, RPS)])
        plsc.subcore_barrier()

        @pl.loop(0, CPW)
        def _(j):
            cix = wid + NW * j

            @pl.when(cix < NCHUNK)
            def _():
                off = pl.multiple_of(cix * K, K)
                pltpu.sync_copy(src_hbm.at[pl.ds(off, K)], sidx)
                pltpu.sync_copy(dst_hbm.at[pl.ds(off, K)], didx)
                pltpu.sync_copy(hp_hbm.at[sidx], rows)
                pltpu.sync_copy(rows, acc.at[didx], add=True)

        plsc.subcore_barrier()
        pltpu.sync_copy(acc.at[pl.ds(r0, RPS)], out_hbm.at[cid, pl.ds(r0, RPS)])

    return prop_kernel(hp, src, dst, zeros64)


def _stage_a_body(x_ref, w1_ref, d0_ref, d1_ref, hp_ref, dinv_ref):
    deg = 1.0 + d0_ref[...] + d1_ref[...]
    dinv = lax.rsqrt(deg)
    h = jnp.dot(x_ref[...], w1_ref[...], preferred_element_type=jnp.float32)
    hp_ref[...] = h * dinv
    dinv_ref[...] = dinv


def _tc_stage_a(x, W1, d0, d1):
    return pl.pallas_call(
        _stage_a_body,
        out_shape=(jax.ShapeDtypeStruct((N, C), jnp.float32),
                   jax.ShapeDtypeStruct((N, 1), jnp.float32)),
    )(x, W1, d0, d1)


def _stage_b_body(hp1_ref, p0_ref, p1_ref, dinv_ref, w2_ref, b1_ref, hp2_ref):
    acc = hp1_ref[...] + p0_ref[...] + p1_ref[...]
    y1 = jnp.maximum(acc * dinv_ref[...] + b1_ref[...], 0.0)
    h2 = jnp.dot(y1, w2_ref[...], preferred_element_type=jnp.float32)
    hp2_ref[...] = h2 * dinv_ref[...]


def _tc_stage_b(hp1, p0, p1, dinv, W2, b1):
    return pl.pallas_call(
        _stage_b_body,
        out_shape=jax.ShapeDtypeStruct((N, C), jnp.float32),
    )(hp1, p0, p1, dinv, W2, b1)


def _stage_c_body(hp2_ref, q0_ref, q1_ref, dinv_ref, b2_ref, batch_ref,
                  fp_ref, lw1a_ref, lw1b_ref, lb1_ref, lw2_ref, lb2_ref,
                  out_ref):
    x1 = (hp2_ref[...] + q0_ref[...] + q1_ref[...]) * dinv_ref[...] + b2_ref[...]
    gids = lax.broadcasted_iota(jnp.int32, (256, N), 0)
    P = (batch_ref[...] == gids).astype(jnp.float32)
    sums = jnp.dot(P, x1, preferred_element_type=jnp.float32)
    cnt = jnp.sum(P, axis=1, keepdims=True)
    pooled = sums / jnp.maximum(cnt, 1.0)
    z1 = (jnp.dot(pooled, lw1a_ref[...], preferred_element_type=jnp.float32)
          + jnp.dot(fp_ref[...], lw1b_ref[...], preferred_element_type=jnp.float32)
          + lb1_ref[...])
    z1 = jnp.maximum(z1, 0.0)
    out_ref[...] = (jnp.dot(z1, lw2_ref[...], preferred_element_type=jnp.float32)
                    + lb2_ref[...])


def _tc_stage_c(hp2, q0, q1, dinv, b2, batch, fp, lw1a, lw1b, lb1, lw2, lb2):
    return pl.pallas_call(
        _stage_c_body,
        out_shape=jax.ShapeDtypeStruct((256, 2), jnp.float32),
    )(hp2, q0, q1, dinv, b2, batch, fp, lw1a, lw1b, lb1, lw2, lb2)


def kernel(x, edge_index, edge_attr, batch_index, mol_fingerprints,
           W1, b1, W2, b2, LW1, LB1, LW2, LB2):
    del edge_attr
    src = edge_index[0]
    dst = edge_index[1]
    zeros64 = jnp.zeros((NP, C), jnp.float32)
    zeros16 = jnp.zeros((NP, DC), jnp.float32)

    degp = _sc_degree(dst, zeros16)
    d0 = degp[0, :N, 0:1]
    d1 = degp[1, :N, 0:1]

    hp1, dinv = _tc_stage_a(x, W1, d0, d1)
    p = _sc_propagate(hp1, src, dst, zeros64)
    hp2 = _tc_stage_b(hp1, p[0, :N], p[1, :N], dinv, W2, b1.reshape(1, C))
    q = _sc_propagate(hp2, src, dst, zeros64)

    out = _tc_stage_c(
        hp2, q[0, :N], q[1, :N], dinv, b2.reshape(1, C),
        batch_index.reshape(1, N), mol_fingerprints,
        LW1[:C], LW1[C:], LB1.reshape(1, -1), LW2, LB2.reshape(1, -1))
    return out
